# Initial kernel scaffold; baseline (speedup 1.0000x reference)
import jax
import jax.numpy as jnp
from jax.experimental import pallas as pl


def kernel(x, edge_index, W1, b1, W2, b2, W3, b3, W4, b4):
    raise NotImplementedError("write your pallas kernel here")


# speedup vs baseline: 22.7398x; 22.7398x over previous
import functools

import jax
import jax.numpy as jnp
from jax import lax
from jax.experimental import pallas as pl
from jax.experimental.pallas import tpu as pltpu
from jax.experimental.pallas import tpu_sc as plsc

NC = 2
NS = 16
NW = NC * NS
B = 128
F = 8


def _sc_mesh():
    return plsc.VectorSubcoreMesh(core_axis_name="c", subcore_axis_name="s")


def _make_deg_kernel(npad, K):
    stripe = npad // NS

    @functools.partial(
        pl.kernel,
        mesh=_sc_mesh(),
        out_type=jax.ShapeDtypeStruct((NC, npad), jnp.float32),
        compiler_params=pltpu.CompilerParams(use_tc_tiling_on_sc=False),
        scratch_types=[
            pltpu.VMEM((K, B), jnp.int32),
            pltpu.VMEM((B,), jnp.float32),
            pltpu.VMEM_SHARED((npad,), jnp.float32),
        ],
    )
    def deg_kernel(dst_hbm, zeros_hbm, out_hbm, idx_v, ones_v, acc_sh):
        cid = lax.axis_index("c")
        sid = lax.axis_index("s")
        wid = sid * NC + cid
        pltpu.sync_copy(zeros_hbm.at[pl.ds(sid * stripe, stripe)],
                        acc_sh.at[pl.ds(sid * stripe, stripe)])
        for i in range(B // 16):
            ones_v[pl.ds(i * 16, 16)] = jnp.ones((16,), jnp.float32)
        pltpu.sync_copy(dst_hbm.at[wid], idx_v)
        plsc.subcore_barrier()

        def body(j, carry):
            pltpu.sync_copy(ones_v, acc_sh.at[idx_v.at[j]], add=True)
            return carry

        lax.fori_loop(0, K, body, 0)
        plsc.subcore_barrier()
        pltpu.sync_copy(acc_sh.at[pl.ds(sid * stripe, stripe)],
                        out_hbm.at[cid, pl.ds(sid * stripe, stripe)])

    return deg_kernel


def _make_msg_kernel(npad, K):
    stripe = npad // NS

    @functools.partial(
        pl.kernel,
        mesh=_sc_mesh(),
        out_type=jax.ShapeDtypeStruct((NC, npad, F), jnp.float32),
        compiler_params=pltpu.CompilerParams(use_tc_tiling_on_sc=False),
        scratch_types=[
            pltpu.VMEM((K, B), jnp.int32),
            pltpu.VMEM((K, B), jnp.int32),
            pltpu.VMEM((B, F), jnp.float32),
            pltpu.VMEM_SHARED((npad, F), jnp.float32),
            pltpu.SemaphoreType.DMA,
        ],
    )
    def msg_kernel(hd_hbm, src_hbm, dst_hbm, zeros_hbm, out_hbm,
                   src_v, dst_v, rows_v, acc_sh, sem):
        cid = lax.axis_index("c")
        sid = lax.axis_index("s")
        wid = sid * NC + cid
        pltpu.sync_copy(zeros_hbm.at[pl.ds(sid * stripe, stripe)],
                        acc_sh.at[pl.ds(sid * stripe, stripe)])
        pltpu.sync_copy(src_hbm.at[wid], src_v)
        pltpu.sync_copy(dst_hbm.at[wid], dst_v)
        plsc.subcore_barrier()

        def body(j, carry):
            pltpu.async_copy(hd_hbm.at[src_v.at[j]], rows_v, sem).wait()
            pltpu.sync_copy(rows_v, acc_sh.at[dst_v.at[j]], add=True)
            return carry

        lax.fori_loop(0, K, body, 0)
        plsc.subcore_barrier()
        pltpu.sync_copy(acc_sh.at[pl.ds(sid * stripe, stripe)],
                        out_hbm.at[cid, pl.ds(sid * stripe, stripe)])

    return msg_kernel


def _mm_body(x_ref, w_ref, o_ref):
    o_ref[...] = jnp.dot(x_ref[...], w_ref[...],
                         preferred_element_type=jnp.float32,
                         precision=lax.Precision.HIGHEST)


def _matmul(xp, W1):
    npad, D = xp.shape
    bm = 1024
    return pl.pallas_call(
        _mm_body,
        grid=(npad // bm,),
        in_specs=[pl.BlockSpec((bm, D), lambda i: (i, 0)),
                  pl.BlockSpec((D, F), lambda i: (0, 0))],
        out_specs=pl.BlockSpec((bm, F), lambda i: (i, 0)),
        out_shape=jax.ShapeDtypeStruct((npad, F), jnp.float32),
    )(xp, W1)


def _scale_body(degp_ref, h_ref, hd_ref):
    deg = degp_ref[0, :] + degp_ref[1, :] + 1.0
    dinv = lax.rsqrt(deg)
    hd_ref[...] = h_ref[...] * dinv[:, None]


def _scale(degp, h):
    npad = h.shape[0]
    bm = 1024
    return pl.pallas_call(
        _scale_body,
        grid=(npad // bm,),
        in_specs=[pl.BlockSpec((NC, bm), lambda i: (0, i)),
                  pl.BlockSpec((bm, F), lambda i: (i, 0))],
        out_specs=pl.BlockSpec((bm, F), lambda i: (i, 0)),
        out_shape=jax.ShapeDtypeStruct((npad, F), jnp.float32),
    )(degp, h)


def _elu(z):
    return jnp.where(z > 0, z, jnp.exp(jnp.minimum(z, 0.0)) - 1.0)


def _head_body(accp_ref, hd_ref, degp_ref, b1_ref, w2_ref, b2_ref,
               w3_ref, b3_ref, w4_ref, b4_ref, o_ref):
    deg = degp_ref[0, :] + degp_ref[1, :] + 1.0
    dinv = lax.rsqrt(deg)[:, None]
    z = dinv * (accp_ref[0] + accp_ref[1] + hd_ref[...]) + b1_ref[...]
    z = _elu(z)
    z = _elu(jnp.dot(z, w2_ref[...], preferred_element_type=jnp.float32,
                     precision=lax.Precision.HIGHEST) + b2_ref[...])
    z = _elu(jnp.dot(z, w3_ref[...], preferred_element_type=jnp.float32,
                     precision=lax.Precision.HIGHEST) + b3_ref[...])
    z = jnp.dot(z, w4_ref[...], preferred_element_type=jnp.float32,
                precision=lax.Precision.HIGHEST) + b4_ref[...]
    m = jnp.max(z, axis=1, keepdims=True)
    lse = m + jnp.log(jnp.sum(jnp.exp(z - m), axis=1, keepdims=True))
    o_ref[...] = z - lse


def _head(accp, hd, degp, b1, W2, b2, W3, b3, W4, b4):
    npad = hd.shape[0]
    bm = 1024
    full = lambda shape: pl.BlockSpec(shape, lambda i: tuple(0 for _ in shape))
    return pl.pallas_call(
        _head_body,
        grid=(npad // bm,),
        in_specs=[pl.BlockSpec((NC, bm, F), lambda i: (0, i, 0)),
                  pl.BlockSpec((bm, F), lambda i: (i, 0)),
                  pl.BlockSpec((NC, bm), lambda i: (0, i)),
                  full(b1.shape), full(W2.shape), full(b2.shape),
                  full(W3.shape), full(b3.shape), full(W4.shape),
                  full(b4.shape)],
        out_specs=pl.BlockSpec((bm, 2), lambda i: (i, 0)),
        out_shape=jax.ShapeDtypeStruct((npad, 2), jnp.float32),
    )(accp, hd, degp, b1, W2, b2, W3, b3, W4, b4)


def kernel(x, edge_index, W1, b1, W2, b2, W3, b3, W4, b4):
    N, D = x.shape
    E = edge_index.shape[1]
    npad = ((N + 1023) // 1024) * 1024
    EB = NW * B
    Epad = ((E + EB - 1) // EB) * EB
    K = Epad // EB

    src = jnp.concatenate(
        [edge_index[0], jnp.zeros((Epad - E,), jnp.int32)]).reshape(NW, K, B)
    dst = jnp.concatenate(
        [edge_index[1], jnp.full((Epad - E,), npad - 1, jnp.int32)]
    ).reshape(NW, K, B)
    xp = jnp.pad(x, ((0, npad - N), (0, 0)))
    zeros1 = jnp.zeros((npad,), jnp.float32)
    zeros2 = jnp.zeros((npad, F), jnp.float32)

    degp = _make_deg_kernel(npad, K)(dst, zeros1)
    h = _matmul(xp, W1)
    hd = _scale(degp, h)
    accp = _make_msg_kernel(npad, K)(hd, src, dst, zeros2)
    out = _head(accp, hd, degp, b1.reshape(1, F), W2, b2.reshape(1, 16),
                W3, b3.reshape(1, 8), W4, b4.reshape(1, 2))
    return out[:N]


# speedup vs baseline: 25.6999x; 1.1302x over previous
import functools

import jax
import jax.numpy as jnp
from jax import lax
from jax.experimental import pallas as pl
from jax.experimental.pallas import tpu as pltpu
from jax.experimental.pallas import tpu_sc as plsc

NC = 2
NS = 16
NW = NC * NS
B = 128
F = 8
NBUF = 4


def _sc_mesh():
    return plsc.VectorSubcoreMesh(core_axis_name="c", subcore_axis_name="s")


def _make_deg_kernel(npad, K):
    stripe = npad // NS

    @functools.partial(
        pl.kernel,
        mesh=_sc_mesh(),
        out_type=jax.ShapeDtypeStruct((NC, npad), jnp.float32),
        compiler_params=pltpu.CompilerParams(use_tc_tiling_on_sc=False),
        scratch_types=[
            pltpu.VMEM((K, B), jnp.int32),
            pltpu.VMEM((B,), jnp.float32),
            pltpu.VMEM_SHARED((npad,), jnp.float32),
            pltpu.SemaphoreType.DMA,
        ],
    )
    def deg_kernel(dst_hbm, zeros_hbm, out_hbm, idx_v, ones_v, acc_sh, sem):
        cid = lax.axis_index("c")
        sid = lax.axis_index("s")
        wid = sid * NC + cid
        pltpu.sync_copy(zeros_hbm.at[pl.ds(sid * stripe, stripe)],
                        acc_sh.at[pl.ds(sid * stripe, stripe)])
        for i in range(B // 16):
            ones_v[pl.ds(i * 16, 16)] = jnp.ones((16,), jnp.float32)
        pltpu.sync_copy(dst_hbm.at[wid], idx_v)
        plsc.subcore_barrier()

        def body(j, carry):
            pltpu.async_copy(ones_v, acc_sh.at[idx_v.at[j]], sem, add=True)
            return carry

        lax.fori_loop(0, K, body, 0)

        def drain(j, carry):
            pltpu.make_async_copy(ones_v, acc_sh.at[idx_v.at[j]], sem).wait()
            return carry

        lax.fori_loop(0, K, drain, 0)
        plsc.subcore_barrier()
        pltpu.sync_copy(acc_sh.at[pl.ds(sid * stripe, stripe)],
                        out_hbm.at[cid, pl.ds(sid * stripe, stripe)])

    return deg_kernel


def _make_msg_kernel(npad, K):
    stripe = npad // NS

    @functools.partial(
        pl.kernel,
        mesh=_sc_mesh(),
        out_type=jax.ShapeDtypeStruct((NC, npad, F), jnp.float32),
        compiler_params=pltpu.CompilerParams(use_tc_tiling_on_sc=False),
        scratch_types=[
            pltpu.VMEM((K, B), jnp.int32),
            pltpu.VMEM((K, B), jnp.int32),
            pltpu.VMEM((NBUF, B, F), jnp.float32),
            pltpu.VMEM_SHARED((npad, F), jnp.float32),
            [pltpu.SemaphoreType.DMA] * NBUF,
        ],
    )
    def msg_kernel(hd_hbm, src_hbm, dst_hbm, zeros_hbm, out_hbm,
                   src_v, dst_v, rows_v, acc_sh, sems):
        cid = lax.axis_index("c")
        sid = lax.axis_index("s")
        wid = sid * NC + cid
        pltpu.sync_copy(zeros_hbm.at[pl.ds(sid * stripe, stripe)],
                        acc_sh.at[pl.ds(sid * stripe, stripe)])
        pltpu.sync_copy(src_hbm.at[wid], src_v)
        pltpu.sync_copy(dst_hbm.at[wid], dst_v)
        plsc.subcore_barrier()

        for b in range(NBUF):
            pltpu.async_copy(hd_hbm.at[src_v.at[b]], rows_v.at[b], sems[b])

        def body(g, carry):
            for b in range(NBUF):
                j = g * NBUF + b
                pltpu.make_async_copy(hd_hbm.at[src_v.at[j]],
                                      rows_v.at[b], sems[b]).wait()
                pltpu.sync_copy(rows_v.at[b], acc_sh.at[dst_v.at[j]],
                                add=True)
                jn = j + NBUF

                @pl.when(jn < K)
                def _():
                    pltpu.async_copy(hd_hbm.at[src_v.at[jn]],
                                     rows_v.at[b], sems[b])
            return carry

        lax.fori_loop(0, K // NBUF, body, 0)
        plsc.subcore_barrier()
        pltpu.sync_copy(acc_sh.at[pl.ds(sid * stripe, stripe)],
                        out_hbm.at[cid, pl.ds(sid * stripe, stripe)])

    return msg_kernel


def _mm_body(x_ref, w_ref, o_ref):
    o_ref[...] = jnp.dot(x_ref[...], w_ref[...],
                         preferred_element_type=jnp.float32,
                         precision=lax.Precision.HIGHEST)


def _matmul(xp, W1):
    npad, D = xp.shape
    bm = 1024
    return pl.pallas_call(
        _mm_body,
        grid=(npad // bm,),
        in_specs=[pl.BlockSpec((bm, D), lambda i: (i, 0)),
                  pl.BlockSpec((D, F), lambda i: (0, 0))],
        out_specs=pl.BlockSpec((bm, F), lambda i: (i, 0)),
        out_shape=jax.ShapeDtypeStruct((npad, F), jnp.float32),
    )(xp, W1)


def _scale_body(degp_ref, h_ref, hd_ref):
    deg = degp_ref[0, :] + degp_ref[1, :] + 1.0
    dinv = lax.rsqrt(deg)
    hd_ref[...] = h_ref[...] * dinv[:, None]


def _scale(degp, h):
    npad = h.shape[0]
    bm = 1024
    return pl.pallas_call(
        _scale_body,
        grid=(npad // bm,),
        in_specs=[pl.BlockSpec((NC, bm), lambda i: (0, i)),
                  pl.BlockSpec((bm, F), lambda i: (i, 0))],
        out_specs=pl.BlockSpec((bm, F), lambda i: (i, 0)),
        out_shape=jax.ShapeDtypeStruct((npad, F), jnp.float32),
    )(degp, h)


def _elu(z):
    return jnp.where(z > 0, z, jnp.exp(jnp.minimum(z, 0.0)) - 1.0)


def _head_body(accp_ref, hd_ref, degp_ref, b1_ref, w2_ref, b2_ref,
               w3_ref, b3_ref, w4_ref, b4_ref, o_ref):
    deg = degp_ref[0, :] + degp_ref[1, :] + 1.0
    dinv = lax.rsqrt(deg)[:, None]
    z = dinv * (accp_ref[0] + accp_ref[1] + hd_ref[...]) + b1_ref[...]
    z = _elu(z)
    z = _elu(jnp.dot(z, w2_ref[...], preferred_element_type=jnp.float32,
                     precision=lax.Precision.HIGHEST) + b2_ref[...])
    z = _elu(jnp.dot(z, w3_ref[...], preferred_element_type=jnp.float32,
                     precision=lax.Precision.HIGHEST) + b3_ref[...])
    z = jnp.dot(z, w4_ref[...], preferred_element_type=jnp.float32,
                precision=lax.Precision.HIGHEST) + b4_ref[...]
    m = jnp.max(z, axis=1, keepdims=True)
    lse = m + jnp.log(jnp.sum(jnp.exp(z - m), axis=1, keepdims=True))
    o_ref[...] = z - lse


def _head(accp, hd, degp, b1, W2, b2, W3, b3, W4, b4):
    npad = hd.shape[0]
    bm = 1024
    full = lambda shape: pl.BlockSpec(shape, lambda i: tuple(0 for _ in shape))
    return pl.pallas_call(
        _head_body,
        grid=(npad // bm,),
        in_specs=[pl.BlockSpec((NC, bm, F), lambda i: (0, i, 0)),
                  pl.BlockSpec((bm, F), lambda i: (i, 0)),
                  pl.BlockSpec((NC, bm), lambda i: (0, i)),
                  full(b1.shape), full(W2.shape), full(b2.shape),
                  full(W3.shape), full(b3.shape), full(W4.shape),
                  full(b4.shape)],
        out_specs=pl.BlockSpec((bm, 2), lambda i: (i, 0)),
        out_shape=jax.ShapeDtypeStruct((npad, 2), jnp.float32),
    )(accp, hd, degp, b1, W2, b2, W3, b3, W4, b4)


def kernel(x, edge_index, W1, b1, W2, b2, W3, b3, W4, b4):
    N, D = x.shape
    E = edge_index.shape[1]
    npad = ((N + 1023) // 1024) * 1024
    EB = NW * B
    Epad = ((E + EB - 1) // EB) * EB
    K = Epad // EB

    src = jnp.concatenate(
        [edge_index[0], jnp.zeros((Epad - E,), jnp.int32)]).reshape(NW, K, B)
    dst = jnp.concatenate(
        [edge_index[1], jnp.full((Epad - E,), npad - 1, jnp.int32)]
    ).reshape(NW, K, B)
    xp = jnp.pad(x, ((0, npad - N), (0, 0)))
    zeros1 = jnp.zeros((npad,), jnp.float32)
    zeros2 = jnp.zeros((npad, F), jnp.float32)

    degp = _make_deg_kernel(npad, K)(dst, zeros1)
    h = _matmul(xp, W1)
    hd = _scale(degp, h)
    accp = _make_msg_kernel(npad, K)(hd, src, dst, zeros2)
    out = _head(accp, hd, degp, b1.reshape(1, F), W2, b2.reshape(1, 16),
                W3, b3.reshape(1, 8), W4, b4.reshape(1, 2))
    return out[:N]


# speedup vs baseline: 34.4362x; 1.3399x over previous
import functools

import jax
import jax.numpy as jnp
from jax import lax
from jax.experimental import pallas as pl
from jax.experimental.pallas import tpu as pltpu
from jax.experimental.pallas import tpu_sc as plsc

NC = 2
NS = 16
NW = NC * NS
B = 128
F = 8
NBUF = 4


def _sc_mesh():
    return plsc.VectorSubcoreMesh(core_axis_name="c", subcore_axis_name="s")


def _make_deg_kernel(npad, K):
    stripe = npad // NS

    @functools.partial(
        pl.kernel,
        mesh=_sc_mesh(),
        out_type=jax.ShapeDtypeStruct((NC, npad), jnp.float32),
        compiler_params=pltpu.CompilerParams(use_tc_tiling_on_sc=False),
        scratch_types=[
            pltpu.VMEM((K, B), jnp.int32),
            pltpu.VMEM((B,), jnp.float32),
            pltpu.VMEM_SHARED((npad,), jnp.float32),
            pltpu.SemaphoreType.DMA,
        ],
    )
    def deg_kernel(ep_hbm, zeros_hbm, out_hbm, idx_v, ones_v, acc_sh, sem):
        cid = lax.axis_index("c")
        sid = lax.axis_index("s")
        wid = sid * NC + cid
        pltpu.sync_copy(zeros_hbm.at[pl.ds(sid * stripe, stripe)],
                        acc_sh.at[pl.ds(sid * stripe, stripe)])
        for i in range(B // 16):
            ones_v[pl.ds(i * 16, 16)] = jnp.ones((16,), jnp.float32)
        pltpu.sync_copy(ep_hbm.at[1, wid], idx_v)
        plsc.subcore_barrier()

        def body(j, carry):
            pltpu.async_copy(ones_v, acc_sh.at[idx_v.at[j]], sem, add=True)
            return carry

        lax.fori_loop(0, K, body, 0)

        def drain(j, carry):
            pltpu.make_async_copy(ones_v, acc_sh.at[idx_v.at[j]], sem).wait()
            return carry

        lax.fori_loop(0, K, drain, 0)
        plsc.subcore_barrier()
        pltpu.sync_copy(acc_sh.at[pl.ds(sid * stripe, stripe)],
                        out_hbm.at[cid, pl.ds(sid * stripe, stripe)])

    return deg_kernel


def _make_msg_kernel(npad, K):
    stripe = npad // NS

    @functools.partial(
        pl.kernel,
        mesh=_sc_mesh(),
        out_type=jax.ShapeDtypeStruct((NC, npad, F), jnp.float32),
        compiler_params=pltpu.CompilerParams(use_tc_tiling_on_sc=False),
        scratch_types=[
            pltpu.VMEM((K, B), jnp.int32),
            pltpu.VMEM((K, B), jnp.int32),
            pltpu.VMEM((NBUF, B, F), jnp.float32),
            pltpu.VMEM_SHARED((npad, F), jnp.float32),
            [pltpu.SemaphoreType.DMA] * NBUF,
        ],
    )
    def msg_kernel(hd_hbm, ep_hbm, zeros_hbm, out_hbm,
                   src_v, dst_v, rows_v, acc_sh, sems):
        cid = lax.axis_index("c")
        sid = lax.axis_index("s")
        wid = sid * NC + cid
        pltpu.sync_copy(zeros_hbm.at[pl.ds(sid * stripe, stripe)],
                        acc_sh.at[pl.ds(sid * stripe, stripe)])
        pltpu.sync_copy(ep_hbm.at[0, wid], src_v)
        pltpu.sync_copy(ep_hbm.at[1, wid], dst_v)
        plsc.subcore_barrier()

        for b in range(NBUF):
            pltpu.async_copy(hd_hbm.at[src_v.at[b]], rows_v.at[b], sems[b])

        def body(g, carry):
            for b in range(NBUF):
                j = g * NBUF + b
                pltpu.make_async_copy(hd_hbm.at[src_v.at[j]],
                                      rows_v.at[b], sems[b]).wait()
                pltpu.sync_copy(rows_v.at[b], acc_sh.at[dst_v.at[j]],
                                add=True)
                jn = j + NBUF

                @pl.when(jn < K)
                def _():
                    pltpu.async_copy(hd_hbm.at[src_v.at[jn]],
                                     rows_v.at[b], sems[b])
            return carry

        lax.fori_loop(0, K // NBUF, body, 0)
        plsc.subcore_barrier()
        pltpu.sync_copy(acc_sh.at[pl.ds(sid * stripe, stripe)],
                        out_hbm.at[cid, pl.ds(sid * stripe, stripe)])

    return msg_kernel


def _mmscale_body(x_ref, w_ref, degp_ref, o_ref):
    h = jnp.dot(x_ref[...], w_ref[...], preferred_element_type=jnp.float32)
    deg = degp_ref[0, :] + degp_ref[1, :] + 1.0
    o_ref[...] = h * lax.rsqrt(deg)[:, None]


def _mmscale(x, W1, degp):
    n, D = x.shape
    bm = 1024
    return pl.pallas_call(
        _mmscale_body,
        grid=(pl.cdiv(n, bm),),
        in_specs=[pl.BlockSpec((bm, D), lambda i: (i, 0)),
                  pl.BlockSpec((D, F), lambda i: (0, 0)),
                  pl.BlockSpec((NC, bm), lambda i: (0, i))],
        out_specs=pl.BlockSpec((bm, F), lambda i: (i, 0)),
        out_shape=jax.ShapeDtypeStruct((n, F), jnp.float32),
    )(x, W1, degp)


def _elu(z):
    return jnp.where(z > 0, z, jnp.exp(jnp.minimum(z, 0.0)) - 1.0)


def _head_body(accp_ref, hd_ref, degp_ref, b1_ref, w2_ref, b2_ref,
               w3_ref, b3_ref, w4_ref, b4_ref, o_ref):
    deg = degp_ref[0, :] + degp_ref[1, :] + 1.0
    dinv = lax.rsqrt(deg)[:, None]
    z = dinv * (accp_ref[0] + accp_ref[1] + hd_ref[...]) + b1_ref[...]
    z = _elu(z)
    z = _elu(jnp.dot(z, w2_ref[...], preferred_element_type=jnp.float32)
             + b2_ref[...])
    z = _elu(jnp.dot(z, w3_ref[...], preferred_element_type=jnp.float32)
             + b3_ref[...])
    z = jnp.dot(z, w4_ref[...], preferred_element_type=jnp.float32) \
        + b4_ref[...]
    m = jnp.max(z, axis=1, keepdims=True)
    lse = m + jnp.log(jnp.sum(jnp.exp(z - m), axis=1, keepdims=True))
    o_ref[...] = z - lse


def _head(accp, hd, degp, b1, W2, b2, W3, b3, W4, b4):
    n = hd.shape[0]
    bm = 1024
    full = lambda shape: pl.BlockSpec(shape, lambda i: tuple(0 for _ in shape))
    return pl.pallas_call(
        _head_body,
        grid=(pl.cdiv(n, bm),),
        in_specs=[pl.BlockSpec((NC, bm, F), lambda i: (0, i, 0)),
                  pl.BlockSpec((bm, F), lambda i: (i, 0)),
                  pl.BlockSpec((NC, bm), lambda i: (0, i)),
                  full(b1.shape), full(W2.shape), full(b2.shape),
                  full(W3.shape), full(b3.shape), full(W4.shape),
                  full(b4.shape)],
        out_specs=pl.BlockSpec((bm, 2), lambda i: (i, 0)),
        out_shape=jax.ShapeDtypeStruct((n, 2), jnp.float32),
    )(accp, hd, degp, b1, W2, b2, W3, b3, W4, b4)


def kernel(x, edge_index, W1, b1, W2, b2, W3, b3, W4, b4):
    N, D = x.shape
    E = edge_index.shape[1]
    npad = ((N + 1023) // 1024) * 1024
    nchunks = E // B
    K = (nchunks + NW - 1) // NW
    trash = N + 8

    ep = edge_index.reshape(2, nchunks, B)
    ep = jnp.pad(ep, ((0, 0), (0, NW * K - nchunks), (0, 0)),
                 constant_values=trash)
    ep = ep.reshape(2, NW, K, B)

    zeros1 = jnp.zeros((npad,), jnp.float32)
    zeros2 = jnp.zeros((npad, F), jnp.float32)

    degp = _make_deg_kernel(npad, K)(ep, zeros1)
    hd = _mmscale(x, W1, degp)
    hdp = jnp.pad(hd, ((0, npad - N), (0, 0)))
    accp = _make_msg_kernel(npad, K)(hdp, ep, zeros2)
    return _head(accp, hd, degp, b1.reshape(1, F), W2, b2.reshape(1, 16),
                 W3, b3.reshape(1, 8), W4, b4.reshape(1, 2))


# speedup vs baseline: 36.7499x; 1.0672x over previous
import functools

import jax
import jax.numpy as jnp
from jax import lax
from jax.experimental import pallas as pl
from jax.experimental.pallas import tpu as pltpu
from jax.experimental.pallas import tpu_sc as plsc

NC = 2
NS = 16
NW = NC * NS
B = 128
F = 8
NBUF = 4
L = 16


def _sc_mesh():
    return plsc.VectorSubcoreMesh(core_axis_name="c", subcore_axis_name="s")


def _make_deg_kernel(npad, K):
    stripe = npad // NS

    @functools.partial(
        pl.kernel,
        mesh=_sc_mesh(),
        out_type=jax.ShapeDtypeStruct((NC, npad), jnp.float32),
        compiler_params=pltpu.CompilerParams(use_tc_tiling_on_sc=False),
        scratch_types=[
            pltpu.VMEM((K, B), jnp.int32),
            pltpu.VMEM((B,), jnp.float32),
            pltpu.VMEM_SHARED((npad,), jnp.float32),
            pltpu.SemaphoreType.DMA,
        ],
    )
    def deg_kernel(ep_hbm, zeros_hbm, out_hbm, idx_v, ones_v, acc_sh, sem):
        cid = lax.axis_index("c")
        sid = lax.axis_index("s")
        wid = sid * NC + cid
        pltpu.sync_copy(zeros_hbm.at[pl.ds(sid * stripe, stripe)],
                        acc_sh.at[pl.ds(sid * stripe, stripe)])
        for i in range(B // L):
            ones_v[pl.ds(i * L, L)] = jnp.ones((L,), jnp.float32)
        pltpu.sync_copy(ep_hbm.at[1, wid], idx_v)
        plsc.subcore_barrier()

        def body(j, carry):
            pltpu.async_copy(ones_v, acc_sh.at[idx_v.at[j]], sem, add=True)
            return carry

        lax.fori_loop(0, K, body, 0)

        def drain(j, carry):
            pltpu.make_async_copy(ones_v, acc_sh.at[idx_v.at[j]], sem).wait()
            return carry

        lax.fori_loop(0, K, drain, 0)
        plsc.subcore_barrier()
        pltpu.sync_copy(acc_sh.at[pl.ds(sid * stripe, stripe)],
                        out_hbm.at[cid, pl.ds(sid * stripe, stripe)])

    return deg_kernel


def _make_scale_kernel(npad):
    seg = npad // NW

    @functools.partial(
        pl.kernel,
        mesh=_sc_mesh(),
        out_type=(jax.ShapeDtypeStruct((npad, F), jnp.float32),
                  jax.ShapeDtypeStruct((npad, F), jnp.float32)),
        compiler_params=pltpu.CompilerParams(use_tc_tiling_on_sc=False),
        scratch_types=[
            pltpu.VMEM((seg,), jnp.float32),
            pltpu.VMEM((seg,), jnp.float32),
            pltpu.VMEM((seg,), jnp.float32),
            pltpu.VMEM((seg, F), jnp.float32),
            pltpu.VMEM((seg, F), jnp.float32),
            pltpu.VMEM((seg, F), jnp.float32),
        ],
    )
    def scale_kernel(degp_hbm, h_hbm, hd_hbm, dvr_hbm,
                     d0_v, d1_v, dinv_v, h_v, hd_v, dvr_v):
        cid = lax.axis_index("c")
        sid = lax.axis_index("s")
        wid = sid * NC + cid
        base = wid * seg
        pltpu.sync_copy(degp_hbm.at[0, pl.ds(base, seg)], d0_v)
        pltpu.sync_copy(degp_hbm.at[1, pl.ds(base, seg)], d1_v)
        pltpu.sync_copy(h_hbm.at[pl.ds(base, seg)], h_v)

        def dbody(i, carry):
            sl = pl.ds(i * L, L)
            d = d0_v[sl] + d1_v[sl] + 1.0
            y = jnp.ones((L,), jnp.float32)
            for k in range(1, 10):
                y = jnp.where(d >= float(2 ** (2 * k - 1)),
                              float(2.0 ** -k), y)
            xh = 0.5 * d
            for _ in range(6):
                y = y * (1.5 - xh * y * y)
            dinv_v[sl] = y
            return carry

        lax.fori_loop(0, seg // L, dbody, 0)

        iota = lax.iota(jnp.int32, L)
        step8 = (iota >= F).astype(jnp.int32)
        col8 = iota - step8 * F

        def ebody(r, carry):
            idx = step8 + 2 * r
            s = plsc.load_gather(dinv_v, [idx])
            hp = plsc.load_gather(h_v, [idx, col8])
            plsc.store_scatter(hd_v, [idx, col8], hp * s)
            plsc.store_scatter(dvr_v, [idx, col8], s)
            return carry

        lax.fori_loop(0, seg // 2, ebody, 0)
        pltpu.sync_copy(hd_v, hd_hbm.at[pl.ds(base, seg)])
        pltpu.sync_copy(dvr_v, dvr_hbm.at[pl.ds(base, seg)])

    return scale_kernel


def _make_msg_kernel(npad, K):
    stripe = npad // NS

    @functools.partial(
        pl.kernel,
        mesh=_sc_mesh(),
        out_type=jax.ShapeDtypeStruct((NC, npad, F), jnp.float32),
        compiler_params=pltpu.CompilerParams(use_tc_tiling_on_sc=False),
        scratch_types=[
            pltpu.VMEM((K, B), jnp.int32),
            pltpu.VMEM((K, B), jnp.int32),
            pltpu.VMEM((NBUF, B, F), jnp.float32),
            pltpu.VMEM_SHARED((npad, F), jnp.float32),
            [pltpu.SemaphoreType.DMA] * NBUF,
        ],
    )
    def msg_kernel(hd_hbm, ep_hbm, zeros_hbm, out_hbm,
                   src_v, dst_v, rows_v, acc_sh, sems):
        cid = lax.axis_index("c")
        sid = lax.axis_index("s")
        wid = sid * NC + cid
        pltpu.sync_copy(zeros_hbm.at[pl.ds(sid * stripe, stripe)],
                        acc_sh.at[pl.ds(sid * stripe, stripe)])
        pltpu.sync_copy(ep_hbm.at[0, wid], src_v)
        pltpu.sync_copy(ep_hbm.at[1, wid], dst_v)
        plsc.subcore_barrier()

        for b in range(NBUF):
            pltpu.async_copy(hd_hbm.at[src_v.at[b]], rows_v.at[b], sems[b])

        def body(g, carry):
            for b in range(NBUF):
                j = g * NBUF + b
                pltpu.make_async_copy(hd_hbm.at[src_v.at[j]],
                                      rows_v.at[b], sems[b]).wait()
                pltpu.sync_copy(rows_v.at[b], acc_sh.at[dst_v.at[j]],
                                add=True)
                jn = j + NBUF

                @pl.when(jn < K)
                def _():
                    pltpu.async_copy(hd_hbm.at[src_v.at[jn]],
                                     rows_v.at[b], sems[b])
            return carry

        lax.fori_loop(0, K // NBUF, body, 0)
        plsc.subcore_barrier()
        pltpu.sync_copy(acc_sh.at[pl.ds(sid * stripe, stripe)],
                        out_hbm.at[cid, pl.ds(sid * stripe, stripe)])

    return msg_kernel


def _scale_tc_body(degp_ref, h_ref, hd_ref, dvr_ref):
    deg = degp_ref[0, :] + degp_ref[1, :] + 1.0
    dinv = lax.rsqrt(deg)[:, None]
    hd_ref[...] = h_ref[...] * dinv
    dvr_ref[...] = jnp.broadcast_to(dinv, h_ref.shape)


def _scale_tc(degp, h):
    npad = h.shape[0]
    bm = 1024
    return pl.pallas_call(
        _scale_tc_body,
        grid=(npad // bm,),
        in_specs=[pl.BlockSpec((NC, bm), lambda i: (0, i)),
                  pl.BlockSpec((bm, F), lambda i: (i, 0))],
        out_specs=[pl.BlockSpec((bm, F), lambda i: (i, 0)),
                   pl.BlockSpec((bm, F), lambda i: (i, 0))],
        out_shape=[jax.ShapeDtypeStruct((npad, F), jnp.float32),
                   jax.ShapeDtypeStruct((npad, F), jnp.float32)],
    )(degp, h)


def _mm_body(x_ref, w_ref, o_ref):
    o_ref[...] = jnp.dot(x_ref[...], w_ref[...],
                         preferred_element_type=jnp.float32)


def _mm(x, W1, npad):
    n, D = x.shape
    bm = 1024
    return pl.pallas_call(
        _mm_body,
        grid=(pl.cdiv(n, bm),),
        in_specs=[pl.BlockSpec((bm, D), lambda i: (i, 0)),
                  pl.BlockSpec((D, F), lambda i: (0, 0))],
        out_specs=pl.BlockSpec((bm, F), lambda i: (i, 0)),
        out_shape=jax.ShapeDtypeStruct((npad, F), jnp.float32),
    )(x, W1)


def _elu(z):
    return jnp.where(z > 0, z, jnp.exp(jnp.minimum(z, 0.0)) - 1.0)


def _headp_body(accp_ref, hd_ref, dvr_ref, b1_ref, w2_ref, b2_ref,
                w3_ref, b3_ref, w4_ref, b4_ref, pswap_ref, o_ref):
    acc = accp_ref[0] + accp_ref[1] + hd_ref[...]
    z = dvr_ref[...] * acc + b1_ref[...]
    z = _elu(z)
    z = _elu(jnp.dot(z, w2_ref[...], preferred_element_type=jnp.float32)
             + b2_ref[...])
    z = _elu(jnp.dot(z, w3_ref[...], preferred_element_type=jnp.float32)
             + b3_ref[...])
    z = jnp.dot(z, w4_ref[...], preferred_element_type=jnp.float32) \
        + b4_ref[...]
    sw = jnp.dot(z, pswap_ref[...], preferred_element_type=jnp.float32)
    m = jnp.maximum(z, sw)
    o_ref[...] = z - (m + jnp.log(jnp.exp(z - m) + jnp.exp(sw - m)))


def _headp(accp_p, hd_p, dvr_p, b1t, W2k, b2t, W3k, b3t, W4k, b4t, Pswap,
           nout):
    bm = 64
    full = lambda shape: pl.BlockSpec(shape, lambda i: tuple(0 for _ in shape))
    return pl.pallas_call(
        _headp_body,
        grid=(pl.cdiv(nout, bm),),
        in_specs=[pl.BlockSpec((NC, bm, 128), lambda i: (0, i, 0)),
                  pl.BlockSpec((bm, 128), lambda i: (i, 0)),
                  pl.BlockSpec((bm, 128), lambda i: (i, 0)),
                  full(b1t.shape), full(W2k.shape), full(b2t.shape),
                  full(W3k.shape), full(b3t.shape), full(W4k.shape),
                  full(b4t.shape), full(Pswap.shape)],
        out_specs=pl.BlockSpec((bm, 32), lambda i: (i, 0)),
        out_shape=jax.ShapeDtypeStruct((nout, 32), jnp.float32),
    )(accp_p, hd_p, dvr_p, b1t, W2k, b2t, W3k, b3t, W4k, b4t, Pswap)


def kernel(x, edge_index, W1, b1, W2, b2, W3, b3, W4, b4):
    N, D = x.shape
    E = edge_index.shape[1]
    npad = ((N + 1023) // 1024) * 1024
    nchunks = E // B
    K = (nchunks + NW - 1) // NW
    trash = N + 8

    ep = edge_index.reshape(2, nchunks, B)
    ep = jnp.pad(ep, ((0, 0), (0, NW * K - nchunks), (0, 0)),
                 constant_values=trash)
    ep = ep.reshape(2, NW, K, B)

    zeros1 = jnp.zeros((npad,), jnp.float32)
    zeros2 = jnp.zeros((npad, F), jnp.float32)

    degp = _make_deg_kernel(npad, K)(ep, zeros1)
    h = _mm(x, W1, npad)
    hd, dvr = _scale_tc(degp, h)
    accp = _make_msg_kernel(npad, K)(hd, ep, zeros2)

    nrows = npad * F // 128
    I16 = jnp.eye(16, dtype=jnp.float32)
    W2k = jnp.kron(I16, W2)
    W3k = jnp.kron(I16, W3)
    W4k = jnp.kron(I16, W4)
    Pswap = jnp.kron(I16, jnp.array([[0.0, 1.0], [1.0, 0.0]], jnp.float32))
    b1t = jnp.tile(b1, 16).reshape(1, 128)
    b2t = jnp.tile(b2, 16).reshape(1, 256)
    b3t = jnp.tile(b3, 16).reshape(1, 128)
    b4t = jnp.tile(b4, 16).reshape(1, 32)

    out = _headp(accp.reshape(NC, nrows, 128),
                 hd.reshape(nrows, 128),
                 dvr.reshape(nrows, 128),
                 b1t, W2k, b2t, W3k, b3t, W4k, b4t, Pswap,
                 N * 2 // 32)
    return out.reshape(N, 2)


# speedup vs baseline: 42.5017x; 1.1565x over previous
import functools

import jax
import jax.numpy as jnp
from jax import lax
from jax.experimental import pallas as pl
from jax.experimental.pallas import tpu as pltpu
from jax.experimental.pallas import tpu_sc as plsc

NC = 2
NS = 16
NW = NC * NS
B = 128
F = 8
NBUF = 4
L = 16
K0 = 52
K1 = 32


def _sc_mesh():
    return plsc.VectorSubcoreMesh(core_axis_name="c", subcore_axis_name="s")


def _chunk_range(cid, sid):
    start = jnp.where(cid == 0, sid * K0, NS * K0 + sid * K1)
    myk = jnp.where(cid == 0, K0, K1)
    return start, myk


def _load_chunks(ep_hbm, row, start, cid, idx_v):
    pltpu.sync_copy(ep_hbm.at[row, pl.ds(start, K1)], idx_v.at[pl.ds(0, K1)])

    @pl.when(cid == 0)
    def _():
        pltpu.sync_copy(ep_hbm.at[row, pl.ds(start + K1, K0 - K1)],
                        idx_v.at[pl.ds(K1, K0 - K1)])


def _make_deg_kernel(npad):
    stripe = npad // NS

    @functools.partial(
        pl.kernel,
        mesh=_sc_mesh(),
        out_type=jax.ShapeDtypeStruct((NC, npad), jnp.float32),
        compiler_params=pltpu.CompilerParams(use_tc_tiling_on_sc=False),
        scratch_types=[
            pltpu.VMEM((K0, B), jnp.int32),
            pltpu.VMEM((B,), jnp.float32),
            pltpu.VMEM_SHARED((npad,), jnp.float32),
            pltpu.SemaphoreType.DMA,
        ],
    )
    def deg_kernel(ep_hbm, zeros_hbm, out_hbm, idx_v, ones_v, acc_sh, sem):
        cid = lax.axis_index("c")
        sid = lax.axis_index("s")
        start, myk = _chunk_range(cid, sid)
        pltpu.sync_copy(zeros_hbm.at[pl.ds(sid * stripe, stripe)],
                        acc_sh.at[pl.ds(sid * stripe, stripe)])
        for i in range(B // L):
            ones_v[pl.ds(i * L, L)] = jnp.ones((L,), jnp.float32)
        _load_chunks(ep_hbm, 1, start, cid, idx_v)
        plsc.subcore_barrier()

        def body(j, carry):
            pltpu.async_copy(ones_v, acc_sh.at[idx_v.at[j]], sem, add=True)
            return carry

        lax.fori_loop(0, myk, body, 0)

        def drain(j, carry):
            pltpu.make_async_copy(ones_v, acc_sh.at[idx_v.at[j]], sem).wait()
            return carry

        lax.fori_loop(0, myk, drain, 0)
        plsc.subcore_barrier()
        pltpu.sync_copy(acc_sh.at[pl.ds(sid * stripe, stripe)],
                        out_hbm.at[cid, pl.ds(sid * stripe, stripe)])

    return deg_kernel


def _make_msg_kernel(npad):
    stripe = npad // NS

    @functools.partial(
        pl.kernel,
        mesh=_sc_mesh(),
        out_type=jax.ShapeDtypeStruct((NC, npad, F), jnp.float32),
        compiler_params=pltpu.CompilerParams(use_tc_tiling_on_sc=False),
        scratch_types=[
            pltpu.VMEM((K0, B), jnp.int32),
            pltpu.VMEM((K0, B), jnp.int32),
            pltpu.VMEM((NBUF, B, F), jnp.float32),
            pltpu.VMEM_SHARED((npad, F), jnp.float32),
            [pltpu.SemaphoreType.DMA] * NBUF,
        ],
    )
    def msg_kernel(hd_hbm, ep_hbm, zeros_hbm, out_hbm,
                   src_v, dst_v, rows_v, acc_sh, sems):
        cid = lax.axis_index("c")
        sid = lax.axis_index("s")
        start, myk = _chunk_range(cid, sid)
        pltpu.sync_copy(zeros_hbm.at[pl.ds(sid * stripe, stripe)],
                        acc_sh.at[pl.ds(sid * stripe, stripe)])
        _load_chunks(ep_hbm, 0, start, cid, src_v)
        _load_chunks(ep_hbm, 1, start, cid, dst_v)
        plsc.subcore_barrier()

        for b in range(NBUF):
            pltpu.async_copy(hd_hbm.at[src_v.at[b]], rows_v.at[b], sems[b])

        def body(g, carry):
            for b in range(NBUF):
                j = g * NBUF + b
                pltpu.make_async_copy(hd_hbm.at[src_v.at[j]],
                                      rows_v.at[b], sems[b]).wait()
                pltpu.sync_copy(rows_v.at[b], acc_sh.at[dst_v.at[j]],
                                add=True)
                jn = j + NBUF

                @pl.when(jn < myk)
                def _():
                    pltpu.async_copy(hd_hbm.at[src_v.at[jn]],
                                     rows_v.at[b], sems[b])
            return carry

        lax.fori_loop(0, myk // NBUF, body, 0)
        plsc.subcore_barrier()
        pltpu.sync_copy(acc_sh.at[pl.ds(sid * stripe, stripe)],
                        out_hbm.at[cid, pl.ds(sid * stripe, stripe)])

    return msg_kernel


def _mmscale_body(x_ref, w_ref, degp_ref, o_ref):
    h = jnp.dot(x_ref[...], w_ref[...], preferred_element_type=jnp.float32)
    deg = degp_ref[0, :] + degp_ref[1, :]
    o_ref[...] = h * lax.rsqrt(deg)[:, None]


def _mmscale(x, W1, degp, npad):
    n, D = x.shape
    bm = 1024
    return pl.pallas_call(
        _mmscale_body,
        grid=(npad // bm,),
        in_specs=[pl.BlockSpec((bm, D), lambda i: (i, 0)),
                  pl.BlockSpec((D, F), lambda i: (0, 0)),
                  pl.BlockSpec((NC, bm), lambda i: (0, i))],
        out_specs=pl.BlockSpec((bm, F), lambda i: (i, 0)),
        out_shape=jax.ShapeDtypeStruct((npad, F), jnp.float32),
    )(x, W1, degp)


def _elu(z):
    return jnp.where(z > 0, z, jnp.exp(jnp.minimum(z, 0.0)) - 1.0)


def _headp_body(accp_ref, deg16_ref, e16_ref, b1_ref, w2_ref, b2_ref,
                w3_ref, b3_ref, w4_ref, b4_ref, pswap_ref, o_ref):
    deg = deg16_ref[0] + deg16_ref[1]
    dvr = jnp.dot(lax.rsqrt(deg), e16_ref[...],
                  preferred_element_type=jnp.float32)
    z = dvr * (accp_ref[0] + accp_ref[1]) + b1_ref[...]
    z = _elu(z)
    z = _elu(jnp.dot(z, w2_ref[...], preferred_element_type=jnp.float32)
             + b2_ref[...])
    z = _elu(jnp.dot(z, w3_ref[...], preferred_element_type=jnp.float32)
             + b3_ref[...])
    z = jnp.dot(z, w4_ref[...], preferred_element_type=jnp.float32) \
        + b4_ref[...]
    sw = jnp.dot(z, pswap_ref[...], preferred_element_type=jnp.float32)
    m = jnp.maximum(z, sw)
    o_ref[...] = z - (m + jnp.log(jnp.exp(z - m) + jnp.exp(sw - m)))


def _headp(accp_p, deg16, E16, b1t, W2k, b2t, W3k, b3t, W4k, b4t, Pswap,
           nout):
    bm = 64
    full = lambda shape: pl.BlockSpec(shape, lambda i: tuple(0 for _ in shape))
    return pl.pallas_call(
        _headp_body,
        grid=(pl.cdiv(nout, bm),),
        in_specs=[pl.BlockSpec((NC, bm, 128), lambda i: (0, i, 0)),
                  pl.BlockSpec((NC, bm, L), lambda i: (0, i, 0)),
                  full(E16.shape), full(b1t.shape), full(W2k.shape),
                  full(b2t.shape), full(W3k.shape), full(b3t.shape),
                  full(W4k.shape), full(b4t.shape), full(Pswap.shape)],
        out_specs=pl.BlockSpec((bm, 32), lambda i: (i, 0)),
        out_shape=jax.ShapeDtypeStruct((nout, 32), jnp.float32),
    )(accp_p, deg16, E16, b1t, W2k, b2t, W3k, b3t, W4k, b4t, Pswap)


def kernel(x, edge_index, W1, b1, W2, b2, W3, b3, W4, b4):
    N, D = x.shape
    E = edge_index.shape[1]
    npad = ((N + 1023) // 1024) * 1024
    trash = N + 8

    loop = jnp.broadcast_to(jnp.arange(npad, dtype=jnp.int32), (2, npad))
    epf = jnp.concatenate([edge_index, loop], axis=1)
    nchunks = epf.shape[1] // B
    total = NS * (K0 + K1)
    ep = epf.reshape(2, nchunks, B)
    ep = jnp.pad(ep, ((0, 0), (0, total - nchunks), (0, 0)),
                 constant_values=trash)

    zeros1 = jnp.zeros((npad,), jnp.float32)
    zeros2 = jnp.zeros((npad, F), jnp.float32)

    degp = _make_deg_kernel(npad)(ep, zeros1)
    hd = _mmscale(x, W1, degp, npad)
    accp = _make_msg_kernel(npad)(hd, ep, zeros2)

    nrows = npad * F // 128
    I16 = jnp.eye(16, dtype=jnp.float32)
    W2k = jnp.kron(I16, W2)
    W3k = jnp.kron(I16, W3)
    W4k = jnp.kron(I16, W4)
    Pswap = jnp.kron(I16, jnp.array([[0.0, 1.0], [1.0, 0.0]], jnp.float32))
    E16 = jnp.kron(I16, jnp.ones((1, F), jnp.float32))
    b1t = jnp.tile(b1, 16).reshape(1, 128)
    b2t = jnp.tile(b2, 16).reshape(1, 256)
    b3t = jnp.tile(b3, 16).reshape(1, 128)
    b4t = jnp.tile(b4, 16).reshape(1, 32)

    out = _headp(accp.reshape(NC, nrows, 128),
                 degp.reshape(NC, npad // L, L),
                 E16, b1t, W2k, b2t, W3k, b3t, W4k, b4t, Pswap,
                 N * 2 // 32)
    return out.reshape(N, 2)


# speedup vs baseline: 42.6849x; 1.0043x over previous
import functools

import jax
import jax.numpy as jnp
from jax import lax
from jax.experimental import pallas as pl
from jax.experimental.pallas import tpu as pltpu
from jax.experimental.pallas import tpu_sc as plsc

NC = 2
NS = 16
NW = NC * NS
B = 128
F = 8
NBUF = 4
L = 16
HEAVY = 1
K0 = 52
K1 = 32


def _sc_mesh():
    return plsc.VectorSubcoreMesh(core_axis_name="c", subcore_axis_name="s")


def _chunk_range(cid, sid):
    start = jnp.where(cid == HEAVY, sid * K0, NS * K0 + sid * K1)
    myk = jnp.where(cid == HEAVY, K0, K1)
    return start, myk


def _load_chunks(ep_hbm, row, start, cid, idx_v):
    pltpu.sync_copy(ep_hbm.at[row, pl.ds(start, K1)], idx_v.at[pl.ds(0, K1)])

    @pl.when(cid == HEAVY)
    def _():
        pltpu.sync_copy(ep_hbm.at[row, pl.ds(start + K1, K0 - K1)],
                        idx_v.at[pl.ds(K1, K0 - K1)])


def _make_deg_kernel(npad):
    stripe = npad // NS

    @functools.partial(
        pl.kernel,
        mesh=_sc_mesh(),
        out_type=jax.ShapeDtypeStruct((NC, npad), jnp.float32),
        compiler_params=pltpu.CompilerParams(use_tc_tiling_on_sc=False),
        scratch_types=[
            pltpu.VMEM((K0, B), jnp.int32),
            pltpu.VMEM((B,), jnp.float32),
            pltpu.VMEM_SHARED((npad,), jnp.float32),
            pltpu.SemaphoreType.DMA,
        ],
    )
    def deg_kernel(ep_hbm, zeros_hbm, out_hbm, idx_v, ones_v, acc_sh, sem):
        cid = lax.axis_index("c")
        sid = lax.axis_index("s")
        start, myk = _chunk_range(cid, sid)
        pltpu.sync_copy(zeros_hbm.at[pl.ds(sid * stripe, stripe)],
                        acc_sh.at[pl.ds(sid * stripe, stripe)])
        for i in range(B // L):
            ones_v[pl.ds(i * L, L)] = jnp.ones((L,), jnp.float32)
        _load_chunks(ep_hbm, 1, start, cid, idx_v)
        plsc.subcore_barrier()

        def body(j, carry):
            pltpu.async_copy(ones_v, acc_sh.at[idx_v.at[j]], sem, add=True)
            return carry

        lax.fori_loop(0, myk, body, 0)

        def drain(j, carry):
            pltpu.make_async_copy(ones_v, acc_sh.at[idx_v.at[j]], sem).wait()
            return carry

        lax.fori_loop(0, myk, drain, 0)
        plsc.subcore_barrier()
        pltpu.sync_copy(acc_sh.at[pl.ds(sid * stripe, stripe)],
                        out_hbm.at[cid, pl.ds(sid * stripe, stripe)])

    return deg_kernel


def _make_msg_kernel(npad):
    stripe = npad // NS

    @functools.partial(
        pl.kernel,
        mesh=_sc_mesh(),
        out_type=jax.ShapeDtypeStruct((NC, npad, F), jnp.float32),
        compiler_params=pltpu.CompilerParams(use_tc_tiling_on_sc=False),
        scratch_types=[
            pltpu.VMEM((K0, B), jnp.int32),
            pltpu.VMEM((K0, B), jnp.int32),
            pltpu.VMEM((NBUF, B, F), jnp.float32),
            pltpu.VMEM_SHARED((npad, F), jnp.float32),
            [pltpu.SemaphoreType.DMA] * NBUF,
        ],
    )
    def msg_kernel(hd_hbm, ep_hbm, zeros_hbm, out_hbm,
                   src_v, dst_v, rows_v, acc_sh, sems):
        cid = lax.axis_index("c")
        sid = lax.axis_index("s")
        start, myk = _chunk_range(cid, sid)
        pltpu.sync_copy(zeros_hbm.at[pl.ds(sid * stripe, stripe)],
                        acc_sh.at[pl.ds(sid * stripe, stripe)])
        _load_chunks(ep_hbm, 0, start, cid, src_v)
        _load_chunks(ep_hbm, 1, start, cid, dst_v)
        plsc.subcore_barrier()

        for b in range(NBUF):
            pltpu.async_copy(hd_hbm.at[src_v.at[b]], rows_v.at[b], sems[b])

        def body(g, carry):
            for b in range(NBUF):
                j = g * NBUF + b
                pltpu.make_async_copy(hd_hbm.at[src_v.at[j]],
                                      rows_v.at[b], sems[b]).wait()
                pltpu.sync_copy(rows_v.at[b], acc_sh.at[dst_v.at[j]],
                                add=True)
                jn = j + NBUF

                @pl.when(jn < myk)
                def _():
                    pltpu.async_copy(hd_hbm.at[src_v.at[jn]],
                                     rows_v.at[b], sems[b])
            return carry

        lax.fori_loop(0, myk // NBUF, body, 0)
        plsc.subcore_barrier()
        pltpu.sync_copy(acc_sh.at[pl.ds(sid * stripe, stripe)],
                        out_hbm.at[cid, pl.ds(sid * stripe, stripe)])

    return msg_kernel


def _mmscale_body(x_ref, w_ref, degp_ref, o_ref):
    h = jnp.dot(x_ref[...], w_ref[...], preferred_element_type=jnp.float32)
    deg = degp_ref[0, :] + degp_ref[1, :]
    o_ref[...] = h * lax.rsqrt(deg)[:, None]


def _mmscale(x, W1, degp, npad):
    n, D = x.shape
    bm = 1024
    return pl.pallas_call(
        _mmscale_body,
        grid=(npad // bm,),
        in_specs=[pl.BlockSpec((bm, D), lambda i: (i, 0)),
                  pl.BlockSpec((D, F), lambda i: (0, 0)),
                  pl.BlockSpec((NC, bm), lambda i: (0, i))],
        out_specs=pl.BlockSpec((bm, F), lambda i: (i, 0)),
        out_shape=jax.ShapeDtypeStruct((npad, F), jnp.float32),
    )(x, W1, degp)


def _elu(z):
    return jnp.where(z > 0, z, jnp.exp(jnp.minimum(z, 0.0)) - 1.0)


def _headp_body(accp_ref, deg16_ref, e16_ref, b1_ref, w2_ref, b2_ref,
                w3_ref, b3_ref, w4_ref, b4_ref, pswap_ref, o_ref):
    deg = deg16_ref[0] + deg16_ref[1]
    dvr = jnp.dot(lax.rsqrt(deg), e16_ref[...],
                  preferred_element_type=jnp.float32)
    z = dvr * (accp_ref[0] + accp_ref[1]) + b1_ref[...]
    z = _elu(z)
    z = _elu(jnp.dot(z, w2_ref[...], preferred_element_type=jnp.float32)
             + b2_ref[...])
    z = _elu(jnp.dot(z, w3_ref[...], preferred_element_type=jnp.float32)
             + b3_ref[...])
    z = jnp.dot(z, w4_ref[...], preferred_element_type=jnp.float32) \
        + b4_ref[...]
    sw = jnp.dot(z, pswap_ref[...], preferred_element_type=jnp.float32)
    m = jnp.maximum(z, sw)
    o_ref[...] = z - (m + jnp.log(jnp.exp(z - m) + jnp.exp(sw - m)))


def _headp(accp_p, deg16, E16, b1t, W2k, b2t, W3k, b3t, W4k, b4t, Pswap,
           nout):
    bm = 64
    full = lambda shape: pl.BlockSpec(shape, lambda i: tuple(0 for _ in shape))
    return pl.pallas_call(
        _headp_body,
        grid=(pl.cdiv(nout, bm),),
        in_specs=[pl.BlockSpec((NC, bm, 128), lambda i: (0, i, 0)),
                  pl.BlockSpec((NC, bm, L), lambda i: (0, i, 0)),
                  full(E16.shape), full(b1t.shape), full(W2k.shape),
                  full(b2t.shape), full(W3k.shape), full(b3t.shape),
                  full(W4k.shape), full(b4t.shape), full(Pswap.shape)],
        out_specs=pl.BlockSpec((bm, 32), lambda i: (i, 0)),
        out_shape=jax.ShapeDtypeStruct((nout, 32), jnp.float32),
    )(accp_p, deg16, E16, b1t, W2k, b2t, W3k, b3t, W4k, b4t, Pswap)


def kernel(x, edge_index, W1, b1, W2, b2, W3, b3, W4, b4):
    N, D = x.shape
    E = edge_index.shape[1]
    npad = ((N + 1023) // 1024) * 1024
    trash = N + 8

    loop = jnp.broadcast_to(jnp.arange(npad, dtype=jnp.int32), (2, npad))
    epf = jnp.concatenate([edge_index, loop], axis=1)
    nchunks = epf.shape[1] // B
    total = NS * (K0 + K1)
    ep = epf.reshape(2, nchunks, B)
    ep = jnp.pad(ep, ((0, 0), (0, total - nchunks), (0, 0)),
                 constant_values=trash)

    zeros1 = jnp.zeros((npad,), jnp.float32)
    zeros2 = jnp.zeros((npad, F), jnp.float32)

    degp = _make_deg_kernel(npad)(ep, zeros1)
    hd = _mmscale(x, W1, degp, npad)
    accp = _make_msg_kernel(npad)(hd, ep, zeros2)

    nrows = npad * F // 128
    I16 = jnp.eye(16, dtype=jnp.float32)
    W2k = jnp.kron(I16, W2)
    W3k = jnp.kron(I16, W3)
    W4k = jnp.kron(I16, W4)
    Pswap = jnp.kron(I16, jnp.array([[0.0, 1.0], [1.0, 0.0]], jnp.float32))
    E16 = jnp.kron(I16, jnp.ones((1, F), jnp.float32))
    b1t = jnp.tile(b1, 16).reshape(1, 128)
    b2t = jnp.tile(b2, 16).reshape(1, 256)
    b3t = jnp.tile(b3, 16).reshape(1, 128)
    b4t = jnp.tile(b4, 16).reshape(1, 32)

    out = _headp(accp.reshape(NC, nrows, 128),
                 degp.reshape(NC, npad // L, L),
                 E16, b1t, W2k, b2t, W3k, b3t, W4k, b4t, Pswap,
                 N * 2 // 32)
    return out.reshape(N, 2)


# speedup vs baseline: 42.7403x; 1.0013x over previous
import functools

import jax
import jax.numpy as jnp
from jax import lax
from jax.experimental import pallas as pl
from jax.experimental.pallas import tpu as pltpu
from jax.experimental.pallas import tpu_sc as plsc

NC = 2
NS = 16
NW = NC * NS
B = 128
F = 8
NBUF = 4
L = 16
HEAVY = 1
K0 = 52
K1 = 32


def _sc_mesh():
    return plsc.VectorSubcoreMesh(core_axis_name="c", subcore_axis_name="s")


def _chunk_range(cid, sid):
    start = jnp.where(cid == HEAVY, sid * K0, NS * K0 + sid * K1)
    myk = jnp.where(cid == HEAVY, K0, K1)
    return start, myk


def _load_chunks(ep_hbm, row, start, cid, idx_v):
    pltpu.sync_copy(ep_hbm.at[row, pl.ds(start, K1)], idx_v.at[pl.ds(0, K1)])

    @pl.when(cid == HEAVY)
    def _():
        pltpu.sync_copy(ep_hbm.at[row, pl.ds(start + K1, K0 - K1)],
                        idx_v.at[pl.ds(K1, K0 - K1)])


def _make_deg_kernel(npad):
    stripe = npad // NS

    @functools.partial(
        pl.kernel,
        mesh=_sc_mesh(),
        out_type=jax.ShapeDtypeStruct((NC, npad), jnp.float32),
        compiler_params=pltpu.CompilerParams(use_tc_tiling_on_sc=False),
        scratch_types=[
            pltpu.VMEM((K0, B), jnp.int32),
            pltpu.VMEM((B,), jnp.float32),
            pltpu.VMEM_SHARED((npad,), jnp.float32),
            pltpu.SemaphoreType.DMA,
        ],
    )
    def deg_kernel(ep_hbm, zeros_hbm, out_hbm, idx_v, ones_v, acc_sh, sem):
        cid = lax.axis_index("c")
        sid = lax.axis_index("s")
        start, myk = _chunk_range(cid, sid)
        pltpu.sync_copy(zeros_hbm.at[pl.ds(sid * stripe, stripe)],
                        acc_sh.at[pl.ds(sid * stripe, stripe)])
        for i in range(B // L):
            ones_v[pl.ds(i * L, L)] = jnp.ones((L,), jnp.float32)
        _load_chunks(ep_hbm, 1, start, cid, idx_v)
        plsc.subcore_barrier()

        def body(j, carry):
            pltpu.async_copy(ones_v, acc_sh.at[idx_v.at[j]], sem, add=True)
            return carry

        lax.fori_loop(0, myk, body, 0)

        def drain(j, carry):
            pltpu.make_async_copy(ones_v, acc_sh.at[idx_v.at[j]], sem).wait()
            return carry

        lax.fori_loop(0, myk, drain, 0)
        plsc.subcore_barrier()
        pltpu.sync_copy(acc_sh.at[pl.ds(sid * stripe, stripe)],
                        out_hbm.at[cid, pl.ds(sid * stripe, stripe)])

    return deg_kernel


def _make_msg_kernel(npad):
    stripe = npad // NS

    @functools.partial(
        pl.kernel,
        mesh=_sc_mesh(),
        out_type=jax.ShapeDtypeStruct((NC, npad, F), jnp.float32),
        compiler_params=pltpu.CompilerParams(use_tc_tiling_on_sc=False),
        scratch_types=[
            pltpu.VMEM((K0, B), jnp.int32),
            pltpu.VMEM((K0, B), jnp.int32),
            pltpu.VMEM((NBUF, B, F), jnp.float32),
            pltpu.VMEM_SHARED((npad, F), jnp.float32),
            [pltpu.SemaphoreType.DMA] * NBUF,
        ],
    )
    def msg_kernel(hd_hbm, ep_hbm, zeros_hbm, out_hbm,
                   src_v, dst_v, rows_v, acc_sh, sems):
        cid = lax.axis_index("c")
        sid = lax.axis_index("s")
        start, myk = _chunk_range(cid, sid)
        pltpu.sync_copy(zeros_hbm.at[pl.ds(sid * stripe, stripe)],
                        acc_sh.at[pl.ds(sid * stripe, stripe)])
        _load_chunks(ep_hbm, 0, start, cid, src_v)
        _load_chunks(ep_hbm, 1, start, cid, dst_v)
        plsc.subcore_barrier()

        for b in range(NBUF):
            pltpu.async_copy(hd_hbm.at[src_v.at[b]], rows_v.at[b], sems[b])

        def body(g, carry):
            for b in range(NBUF):
                j = g * NBUF + b
                pltpu.make_async_copy(hd_hbm.at[src_v.at[j]],
                                      rows_v.at[b], sems[b]).wait()
                pltpu.sync_copy(rows_v.at[b], acc_sh.at[dst_v.at[j]],
                                add=True)
                jn = j + NBUF

                @pl.when(jn < myk)
                def _():
                    pltpu.async_copy(hd_hbm.at[src_v.at[jn]],
                                     rows_v.at[b], sems[b])
            return carry

        lax.fori_loop(0, myk // NBUF, body, 0)
        plsc.subcore_barrier()
        pltpu.sync_copy(acc_sh.at[pl.ds(sid * stripe, stripe)],
                        out_hbm.at[cid, pl.ds(sid * stripe, stripe)])

    return msg_kernel


def _mmscale_body(x_ref, w_ref, degp_ref, o_ref):
    h = jnp.dot(x_ref[...].astype(jnp.bfloat16),
                w_ref[...].astype(jnp.bfloat16),
                preferred_element_type=jnp.float32)
    deg = degp_ref[0, :] + degp_ref[1, :]
    o_ref[...] = h * lax.rsqrt(deg)[:, None]


def _mmscale(x, W1, degp, npad):
    n, D = x.shape
    bm = 1024
    return pl.pallas_call(
        _mmscale_body,
        grid=(npad // bm,),
        in_specs=[pl.BlockSpec((bm, D), lambda i: (i, 0)),
                  pl.BlockSpec((D, F), lambda i: (0, 0)),
                  pl.BlockSpec((NC, bm), lambda i: (0, i))],
        out_specs=pl.BlockSpec((bm, F), lambda i: (i, 0)),
        out_shape=jax.ShapeDtypeStruct((npad, F), jnp.float32),
    )(x, W1, degp)


def _elu(z):
    return jnp.where(z > 0, z, jnp.exp(jnp.minimum(z, 0.0)) - 1.0)


def _headp_body(accp_ref, deg16_ref, e16_ref, b1_ref, w2_ref, b2_ref,
                w3_ref, b3_ref, w4_ref, b4_ref, pswap_ref, o_ref):
    deg = deg16_ref[0] + deg16_ref[1]
    dvr = jnp.dot(lax.rsqrt(deg), e16_ref[...],
                  preferred_element_type=jnp.float32)
    z = dvr * (accp_ref[0] + accp_ref[1]) + b1_ref[...]
    z = _elu(z)
    z = _elu(jnp.dot(z, w2_ref[...], preferred_element_type=jnp.float32)
             + b2_ref[...])
    z = _elu(jnp.dot(z, w3_ref[...], preferred_element_type=jnp.float32)
             + b3_ref[...])
    z = jnp.dot(z, w4_ref[...], preferred_element_type=jnp.float32) \
        + b4_ref[...]
    sw = jnp.dot(z, pswap_ref[...], preferred_element_type=jnp.float32)
    m = jnp.maximum(z, sw)
    o_ref[...] = z - (m + jnp.log(jnp.exp(z - m) + jnp.exp(sw - m)))


def _headp(accp_p, deg16, E16, b1t, W2k, b2t, W3k, b3t, W4k, b4t, Pswap,
           nout):
    bm = 64
    full = lambda shape: pl.BlockSpec(shape, lambda i: tuple(0 for _ in shape))
    return pl.pallas_call(
        _headp_body,
        grid=(pl.cdiv(nout, bm),),
        in_specs=[pl.BlockSpec((NC, bm, 128), lambda i: (0, i, 0)),
                  pl.BlockSpec((NC, bm, L), lambda i: (0, i, 0)),
                  full(E16.shape), full(b1t.shape), full(W2k.shape),
                  full(b2t.shape), full(W3k.shape), full(b3t.shape),
                  full(W4k.shape), full(b4t.shape), full(Pswap.shape)],
        out_specs=pl.BlockSpec((bm, 32), lambda i: (i, 0)),
        out_shape=jax.ShapeDtypeStruct((nout, 32), jnp.float32),
    )(accp_p, deg16, E16, b1t, W2k, b2t, W3k, b3t, W4k, b4t, Pswap)


def kernel(x, edge_index, W1, b1, W2, b2, W3, b3, W4, b4):
    N, D = x.shape
    E = edge_index.shape[1]
    npad = ((N + 1023) // 1024) * 1024
    trash = N + 8

    loop = jnp.broadcast_to(jnp.arange(npad, dtype=jnp.int32), (2, npad))
    epf = jnp.concatenate([edge_index, loop], axis=1)
    nchunks = epf.shape[1] // B
    total = NS * (K0 + K1)
    ep = epf.reshape(2, nchunks, B)
    ep = jnp.pad(ep, ((0, 0), (0, total - nchunks), (0, 0)),
                 constant_values=trash)

    zeros1 = jnp.zeros((npad,), jnp.float32)
    zeros2 = jnp.zeros((npad, F), jnp.float32)

    degp = _make_deg_kernel(npad)(ep, zeros1)
    hd = _mmscale(x, W1, degp, npad)
    accp = _make_msg_kernel(npad)(hd, ep, zeros2)

    nrows = npad * F // 128
    I16 = jnp.eye(16, dtype=jnp.float32)
    W2k = jnp.kron(I16, W2)
    W3k = jnp.kron(I16, W3)
    W4k = jnp.kron(I16, W4)
    Pswap = jnp.kron(I16, jnp.array([[0.0, 1.0], [1.0, 0.0]], jnp.float32))
    E16 = jnp.kron(I16, jnp.ones((1, F), jnp.float32))
    b1t = jnp.tile(b1, 16).reshape(1, 128)
    b2t = jnp.tile(b2, 16).reshape(1, 256)
    b3t = jnp.tile(b3, 16).reshape(1, 128)
    b4t = jnp.tile(b4, 16).reshape(1, 32)

    out = _headp(accp.reshape(NC, nrows, 128),
                 degp.reshape(NC, npad // L, L),
                 E16, b1t, W2k, b2t, W3k, b3t, W4k, b4t, Pswap,
                 N * 2 // 32)
    return out.reshape(N, 2)
